# baseline (speedup 1.0000x reference)
import functools
import jax
import jax.numpy as jnp
from jax import lax
from jax.experimental import pallas as pl
from jax.experimental.pallas import tpu as pltpu
from jax.experimental.pallas import tpu_sc as plsc

N = 10000
E = 320000
DH = 128
NC = 2
NS = 16
K = 80
NCHUNK_AGG = (E // NS) // K
NCHUNK_DEG = (E // (NS * NC)) // K
SB = 25
NSB_AGG = NCHUNK_AGG // SB
NSB_DEG = NCHUNK_DEG // SB
NPAD = 10240
ROWS_PT = NPAD // NS
ZR = 128

_mesh = plsc.VectorSubcoreMesh(core_axis_name="c", subcore_axis_name="s")


def _deg_body(dst_hbm, out_hbm, dst_v, buf, acc, ssem):
    c = lax.axis_index("c")
    s = lax.axis_index("s")
    zeros16 = jnp.zeros((16,), jnp.float32)
    ones16 = jnp.ones((16,), jnp.float32)

    def fillrow(r, carry):
        for k in range(DH // 16):
            buf[0, r, pl.ds(k * 16, 16)] = zeros16
            buf[1, r, pl.ds(k * 16, 16)] = ones16
        return carry
    lax.fori_loop(0, K, fillrow, 0)

    row0 = s * ROWS_PT
    for q in range(ROWS_PT // K):
        pltpu.sync_copy(buf.at[0], acc.at[pl.ds(row0 + q * K, K)])
    plsc.subcore_barrier()

    wid = s * NC + c

    def sblock(sb, carry):
        pltpu.sync_copy(dst_hbm.at[wid, sb], dst_v)

        def step(j, carry2):
            pltpu.async_copy(buf.at[1], acc.at[dst_v.at[j]], ssem, add=True)
            pltpu.make_async_copy(buf.at[1], acc.at[dst_v.at[j]], ssem).wait()
            return carry2
        lax.fori_loop(0, SB, step, 0)
        return carry
    lax.fori_loop(0, NSB_DEG, sblock, 0)

    plsc.subcore_barrier()
    pltpu.sync_copy(acc.at[pl.ds(row0, ROWS_PT)],
                    out_hbm.at[pl.ds(c * NPAD + row0, ROWS_PT)])


_deg_call = functools.partial(
    pl.kernel,
    out_type=jax.ShapeDtypeStruct((NC * NPAD, DH), jnp.float32),
    mesh=_mesh,
    scratch_types=[
        pltpu.VMEM((SB, K), jnp.int32),
        pltpu.VMEM((2, K, DH), jnp.float32),
        pltpu.VMEM_SHARED((NPAD, DH), jnp.float32),
        pltpu.SemaphoreType.DMA,
    ],
)(_deg_body)


def _agg_body(src_hbm, dst_hbm, g_hbm, out_hbm, src_v, dst_v, buf, acc,
              gsem0, gsem1, ssem0, ssem1):
    c = lax.axis_index("c")
    s = lax.axis_index("s")
    zeros16 = jnp.zeros((16,), jnp.float32)

    def zrow(r, carry):
        for k in range(DH // 16):
            buf[0, r, pl.ds(k * 16, 16)] = zeros16
        return carry
    lax.fori_loop(0, K, zrow, 0)

    row0 = s * ROWS_PT
    for q in range(ROWS_PT // K):
        pltpu.sync_copy(buf.at[0], acc.at[pl.ds(row0 + q * K, K)])

    off = c * NPAD
    plsc.subcore_barrier()

    def gather(j, b, sem):
        pltpu.async_copy(g_hbm.at[src_v.at[j]], buf.at[b], sem)

    def gwait(j, b, sem):
        pltpu.make_async_copy(g_hbm.at[src_v.at[j]], buf.at[b], sem).wait()

    def sissue(j, b, sem):
        pltpu.async_copy(buf.at[b], acc.at[dst_v.at[j]], sem, add=True)

    def swait(j, b, sem):
        pltpu.make_async_copy(buf.at[b], acc.at[dst_v.at[j]], sem).wait()

    def sblock(sb, carry):
        pltpu.sync_copy(src_hbm.at[s, sb], src_v)
        pltpu.sync_copy(dst_hbm.at[s, sb], dst_v)

        def addoff(r, carry2):
            for k in range(K // 16):
                sl = pl.ds(k * 16, 16)
                src_v[r, sl] = src_v[r, sl] + off
            return carry2
        lax.fori_loop(0, SB, addoff, 0)

        gather(0, 0, gsem0)
        gather(1, 1, gsem1)

        def pair(i, carry2):
            j0 = i * 2
            gwait(j0, 0, gsem0)
            sissue(j0, 0, ssem0)
            gwait(j0 + 1, 1, gsem1)
            sissue(j0 + 1, 1, ssem1)
            swait(j0, 0, ssem0)
            gather(jnp.minimum(j0 + 2, SB - 1), 0, gsem0)
            swait(j0 + 1, 1, ssem1)
            gather(jnp.minimum(j0 + 3, SB - 1), 1, gsem1)
            return carry2
        lax.fori_loop(0, SB // 2, pair, 0)

        gwait(SB - 1, 0, gsem0)
        sissue(SB - 1, 0, ssem0)
        gwait(SB - 1, 1, gsem1)
        swait(SB - 1, 0, ssem0)
        return carry
    lax.fori_loop(0, NSB_AGG, sblock, 0)

    plsc.subcore_barrier()
    pltpu.sync_copy(acc.at[pl.ds(row0, ROWS_PT)],
                    out_hbm.at[pl.ds(c * NPAD + row0, ROWS_PT)])


_agg_call = functools.partial(
    pl.kernel,
    out_type=jax.ShapeDtypeStruct((NC * NPAD, DH), jnp.float32),
    mesh=_mesh,
    scratch_types=[
        pltpu.VMEM((SB, K), jnp.int32),
        pltpu.VMEM((SB, K), jnp.int32),
        pltpu.VMEM((2, K, DH), jnp.float32),
        pltpu.VMEM_SHARED((NPAD, DH), jnp.float32),
        pltpu.SemaphoreType.DMA,
        pltpu.SemaphoreType.DMA,
        pltpu.SemaphoreType.DMA,
        pltpu.SemaphoreType.DMA,
    ],
)(_agg_body)


R = 400
NB = N // R


def _dinv_block(deg_ref):
    d = deg_ref[0, :, 0:1] + deg_ref[1, :, 0:1] + 1.0
    return jnp.where(d > 0, lax.rsqrt(jnp.maximum(d, 1e-12)), 0.0)


def _mm1_body(deg_ref, x_ref, w_ref, o_ref):
    dinv = _dinv_block(deg_ref)
    h = jnp.dot(x_ref[...], w_ref[...], preferred_element_type=jnp.float32)
    g = h * dinv
    o_ref[0] = g[:, :DH]
    o_ref[1] = g[:, DH:]


def _layer_body(deg_ref, agg_ref, g_ref, b_ref, w_ref, o_ref):
    dinv = _dinv_block(deg_ref)
    x0 = jnp.maximum((agg_ref[0] + g_ref[0]) * dinv + b_ref[0, :DH], 0.0)
    x1 = jnp.maximum((agg_ref[1] + g_ref[1]) * dinv + b_ref[0, DH:], 0.0)
    x = jnp.concatenate([x0, x1], axis=1)
    h = jnp.dot(x, w_ref[...], preferred_element_type=jnp.float32)
    g = h * dinv
    o_ref[0] = g[:, :DH]
    o_ref[1] = g[:, DH:]


def _final_body(deg_ref, agg_ref, g_ref, b_ref, o_ref):
    dinv = _dinv_block(deg_ref)
    x0 = (agg_ref[0] + g_ref[0]) * dinv + b_ref[0, :DH]
    x1 = (agg_ref[1] + g_ref[1]) * dinv + b_ref[0, DH:]
    o_ref[...] = jnp.concatenate([x0, x1], axis=1)


_deg_spec = pl.BlockSpec((2, R, DH), lambda j: (0, j, 0))
_half_spec = pl.BlockSpec((2, R, DH), lambda j: (0, j, 0))
_b_spec = pl.BlockSpec((1, 2 * DH), lambda j: (0, 0))


def _mm1(deg2, x, W1):
    return pl.pallas_call(
        _mm1_body,
        grid=(NB,),
        in_specs=[
            _deg_spec,
            pl.BlockSpec((R, 128), lambda j: (j, 0)),
            pl.BlockSpec((128, 2 * DH), lambda j: (0, 0)),
        ],
        out_specs=_half_spec,
        out_shape=jax.ShapeDtypeStruct((2, NPAD, DH), jnp.float32),
    )(deg2, x, W1)


def _layer(deg2, agg, g, b, W):
    return pl.pallas_call(
        _layer_body,
        grid=(NB,),
        in_specs=[
            _deg_spec,
            _half_spec,
            _half_spec,
            _b_spec,
            pl.BlockSpec((2 * DH, 2 * DH), lambda j: (0, 0)),
        ],
        out_specs=_half_spec,
        out_shape=jax.ShapeDtypeStruct((2, NPAD, DH), jnp.float32),
    )(deg2, agg, g, b, W)


def _final(deg2, agg, g, b):
    return pl.pallas_call(
        _final_body,
        grid=(NB,),
        in_specs=[_deg_spec, _half_spec, _half_spec, _b_spec],
        out_specs=pl.BlockSpec((R, 2 * DH), lambda j: (j, 0)),
        out_shape=jax.ShapeDtypeStruct((N, 2 * DH), jnp.float32),
    )(deg2, agg, g, b)


def kernel(x, edge_index, W1, b1, W2, b2, W3, b3):
    src_r = edge_index[0].reshape(NS, NSB_AGG, SB, K)
    dst_r = edge_index[1].reshape(NS, NSB_AGG, SB, K)
    b1r = b1.reshape(1, 2 * DH)
    b2r = b2.reshape(1, 2 * DH)
    b3r = b3.reshape(1, 2 * DH)

    dst_d = edge_index[1].reshape(NS * NC, NSB_DEG, SB, K)
    deg2 = _deg_call(dst_d).reshape(2, NPAD, DH)

    g1 = _mm1(deg2, x, W1)
    agg1 = _agg_call(src_r, dst_r, g1.reshape(NC * NPAD, DH)).reshape(2, NPAD, DH)
    g2 = _layer(deg2, agg1, g1, b1r, W2)
    agg2 = _agg_call(src_r, dst_r, g2.reshape(NC * NPAD, DH)).reshape(2, NPAD, DH)
    g3 = _layer(deg2, agg2, g2, b2r, W3)
    agg3 = _agg_call(src_r, dst_r, g3.reshape(NC * NPAD, DH)).reshape(2, NPAD, DH)
    return _final(deg2, agg3, g3, b3r)


# speedup vs baseline: 1.0550x; 1.0550x over previous
import functools
import jax
import jax.numpy as jnp
from jax import lax
from jax.experimental import pallas as pl
from jax.experimental.pallas import tpu as pltpu
from jax.experimental.pallas import tpu_sc as plsc

N = 10000
E = 320000
DH = 128
NC = 2
NS = 16
K = 128
EPT = 20480
EP = NS * EPT
NCHUNK_AGG = EPT // K
SB = 32
NSB_AGG = NCHUNK_AGG // SB
SB_DEG = 16
NCHUNK_DEG = (EP // (NS * NC)) // K
NSB_DEG = NCHUNK_DEG // SB_DEG
NPAD = 10240
ROWS_PT = NPAD // NS

_mesh = plsc.VectorSubcoreMesh(core_axis_name="c", subcore_axis_name="s")


def _deg_body(dst_hbm, out_hbm, dst_v, buf, acc, ssem):
    c = lax.axis_index("c")
    s = lax.axis_index("s")
    zeros16 = jnp.zeros((16,), jnp.float32)
    ones16 = jnp.ones((16,), jnp.float32)

    def fillrow(r, carry):
        for k in range(DH // 16):
            buf[0, r, pl.ds(k * 16, 16)] = zeros16
            buf[1, r, pl.ds(k * 16, 16)] = ones16
        return carry
    lax.fori_loop(0, K, fillrow, 0)

    row0 = s * ROWS_PT
    for q in range(ROWS_PT // K):
        pltpu.sync_copy(buf.at[0], acc.at[pl.ds(row0 + q * K, K)])
    plsc.subcore_barrier()

    wid = s * NC + c

    def sblock(sb, carry):
        pltpu.sync_copy(dst_hbm.at[wid, sb], dst_v)

        def step(j, carry2):
            pltpu.async_copy(buf.at[1], acc.at[dst_v.at[j]], ssem, add=True)
            pltpu.make_async_copy(buf.at[1], acc.at[dst_v.at[j]], ssem).wait()
            return carry2
        lax.fori_loop(0, SB_DEG, step, 0)
        return carry
    lax.fori_loop(0, NSB_DEG, sblock, 0)

    plsc.subcore_barrier()
    pltpu.sync_copy(acc.at[pl.ds(row0, ROWS_PT)],
                    out_hbm.at[pl.ds(c * NPAD + row0, ROWS_PT)])


_deg_call = functools.partial(
    pl.kernel,
    out_type=jax.ShapeDtypeStruct((NC * NPAD, DH), jnp.float32),
    mesh=_mesh,
    scratch_types=[
        pltpu.VMEM((SB_DEG, K), jnp.int32),
        pltpu.VMEM((2, K, DH), jnp.float32),
        pltpu.VMEM_SHARED((NPAD, DH), jnp.float32),
        pltpu.SemaphoreType.DMA,
    ],
)(_deg_body)


def _agg_body(src_hbm, dst_hbm, g_hbm, out_hbm, src_v, dst_v, buf, acc,
              gsem0, gsem1, ssem0, ssem1):
    c = lax.axis_index("c")
    s = lax.axis_index("s")
    zeros16 = jnp.zeros((16,), jnp.float32)

    def zrow(r, carry):
        for k in range(DH // 16):
            buf[0, r, pl.ds(k * 16, 16)] = zeros16
        return carry
    lax.fori_loop(0, K, zrow, 0)

    row0 = s * ROWS_PT
    for q in range(ROWS_PT // K):
        pltpu.sync_copy(buf.at[0], acc.at[pl.ds(row0 + q * K, K)])

    off = c * NPAD
    plsc.subcore_barrier()

    def gather(j, b, sem):
        pltpu.async_copy(g_hbm.at[src_v.at[j]], buf.at[b], sem)

    def gwait(j, b, sem):
        pltpu.make_async_copy(g_hbm.at[src_v.at[j]], buf.at[b], sem).wait()

    def sissue(j, b, sem):
        pltpu.async_copy(buf.at[b], acc.at[dst_v.at[j]], sem, add=True)

    def swait(j, b, sem):
        pltpu.make_async_copy(buf.at[b], acc.at[dst_v.at[j]], sem).wait()

    def sblock(sb, carry):
        pltpu.sync_copy(src_hbm.at[s, sb], src_v)
        pltpu.sync_copy(dst_hbm.at[s, sb], dst_v)

        def addoff(r, carry2):
            for k in range(K // 16):
                sl = pl.ds(k * 16, 16)
                src_v[r, sl] = src_v[r, sl] + off
            return carry2
        lax.fori_loop(0, SB, addoff, 0)

        gather(0, 0, gsem0)
        gather(1, 1, gsem1)

        def pair(i, carry2):
            j0 = i * 2
            gwait(j0, 0, gsem0)
            sissue(j0, 0, ssem0)
            gwait(j0 + 1, 1, gsem1)
            sissue(j0 + 1, 1, ssem1)
            swait(j0, 0, ssem0)
            gather(jnp.minimum(j0 + 2, SB - 1), 0, gsem0)
            swait(j0 + 1, 1, ssem1)
            gather(jnp.minimum(j0 + 3, SB - 1), 1, gsem1)
            return carry2
        lax.fori_loop(0, SB // 2, pair, 0)

        gwait(SB - 1, 0, gsem0)
        gwait(SB - 1, 1, gsem1)
        return carry
    lax.fori_loop(0, NSB_AGG, sblock, 0)

    plsc.subcore_barrier()
    pltpu.sync_copy(acc.at[pl.ds(row0, ROWS_PT)],
                    out_hbm.at[pl.ds(c * NPAD + row0, ROWS_PT)])


_agg_call = functools.partial(
    pl.kernel,
    out_type=jax.ShapeDtypeStruct((NC * NPAD, DH), jnp.float32),
    mesh=_mesh,
    scratch_types=[
        pltpu.VMEM((SB, K), jnp.int32),
        pltpu.VMEM((SB, K), jnp.int32),
        pltpu.VMEM((2, K, DH), jnp.float32),
        pltpu.VMEM_SHARED((NPAD, DH), jnp.float32),
        pltpu.SemaphoreType.DMA,
        pltpu.SemaphoreType.DMA,
        pltpu.SemaphoreType.DMA,
        pltpu.SemaphoreType.DMA,
    ],
)(_agg_body)


R = 400
NB = N // R


def _dinv_block(deg_ref):
    d = deg_ref[0, :, 0:1] + deg_ref[1, :, 0:1] + 1.0
    return jnp.where(d > 0, lax.rsqrt(jnp.maximum(d, 1e-12)), 0.0)


def _mm1_body(deg_ref, x_ref, w_ref, o_ref):
    dinv = _dinv_block(deg_ref)
    h = jnp.dot(x_ref[...], w_ref[...], preferred_element_type=jnp.float32)
    g = h * dinv
    o_ref[0] = g[:, :DH]
    o_ref[1] = g[:, DH:]


def _layer_body(deg_ref, agg_ref, g_ref, b_ref, w_ref, o_ref):
    dinv = _dinv_block(deg_ref)
    x0 = jnp.maximum((agg_ref[0] + g_ref[0]) * dinv + b_ref[0, :DH], 0.0)
    x1 = jnp.maximum((agg_ref[1] + g_ref[1]) * dinv + b_ref[0, DH:], 0.0)
    x = jnp.concatenate([x0, x1], axis=1)
    h = jnp.dot(x, w_ref[...], preferred_element_type=jnp.float32)
    g = h * dinv
    o_ref[0] = g[:, :DH]
    o_ref[1] = g[:, DH:]


def _final_body(deg_ref, agg_ref, g_ref, b_ref, o_ref):
    dinv = _dinv_block(deg_ref)
    x0 = (agg_ref[0] + g_ref[0]) * dinv + b_ref[0, :DH]
    x1 = (agg_ref[1] + g_ref[1]) * dinv + b_ref[0, DH:]
    o_ref[...] = jnp.concatenate([x0, x1], axis=1)


_deg_spec = pl.BlockSpec((2, R, DH), lambda j: (0, j, 0))
_half_spec = pl.BlockSpec((2, R, DH), lambda j: (0, j, 0))
_b_spec = pl.BlockSpec((1, 2 * DH), lambda j: (0, 0))


def _mm1(deg2, x, W1):
    return pl.pallas_call(
        _mm1_body,
        grid=(NB,),
        in_specs=[
            _deg_spec,
            pl.BlockSpec((R, 128), lambda j: (j, 0)),
            pl.BlockSpec((128, 2 * DH), lambda j: (0, 0)),
        ],
        out_specs=_half_spec,
        out_shape=jax.ShapeDtypeStruct((2, NPAD, DH), jnp.float32),
    )(deg2, x, W1)


def _layer(deg2, agg, g, b, W):
    return pl.pallas_call(
        _layer_body,
        grid=(NB,),
        in_specs=[
            _deg_spec,
            _half_spec,
            _half_spec,
            _b_spec,
            pl.BlockSpec((2 * DH, 2 * DH), lambda j: (0, 0)),
        ],
        out_specs=_half_spec,
        out_shape=jax.ShapeDtypeStruct((2, NPAD, DH), jnp.float32),
    )(deg2, agg, g, b, W)


def _final(deg2, agg, g, b):
    return pl.pallas_call(
        _final_body,
        grid=(NB,),
        in_specs=[_deg_spec, _half_spec, _half_spec, _b_spec],
        out_specs=pl.BlockSpec((R, 2 * DH), lambda j: (j, 0)),
        out_shape=jax.ShapeDtypeStruct((N, 2 * DH), jnp.float32),
    )(deg2, agg, g, b)


def kernel(x, edge_index, W1, b1, W2, b2, W3, b3):
    npad_e = EP - E
    pad_src = jnp.arange(npad_e, dtype=jnp.int32) % N
    pad_dst = N + jnp.arange(npad_e, dtype=jnp.int32) % (NPAD - N)
    src_p = jnp.concatenate([edge_index[0], pad_src])
    dst_p = jnp.concatenate([edge_index[1], pad_dst])
    src_r = src_p.reshape(NS, NSB_AGG, SB, K)
    dst_r = dst_p.reshape(NS, NSB_AGG, SB, K)
    b1r = b1.reshape(1, 2 * DH)
    b2r = b2.reshape(1, 2 * DH)
    b3r = b3.reshape(1, 2 * DH)

    dst_d = dst_p.reshape(NS * NC, NSB_DEG, SB_DEG, K)
    deg2 = _deg_call(dst_d).reshape(2, NPAD, DH)

    g1 = _mm1(deg2, x, W1)
    agg1 = _agg_call(src_r, dst_r, g1.reshape(NC * NPAD, DH)).reshape(2, NPAD, DH)
    g2 = _layer(deg2, agg1, g1, b1r, W2)
    agg2 = _agg_call(src_r, dst_r, g2.reshape(NC * NPAD, DH)).reshape(2, NPAD, DH)
    g3 = _layer(deg2, agg2, g2, b2r, W3)
    agg3 = _agg_call(src_r, dst_r, g3.reshape(NC * NPAD, DH)).reshape(2, NPAD, DH)
    return _final(deg2, agg3, g3, b3r)


# speedup vs baseline: 1.0580x; 1.0029x over previous
import functools
import jax
import jax.numpy as jnp
from jax import lax
from jax.experimental import pallas as pl
from jax.experimental.pallas import tpu as pltpu
from jax.experimental.pallas import tpu_sc as plsc

N = 10000
E = 320000
DH = 128
NC = 2
NS = 16
K = 128
EPT = 20480
EP = NS * EPT
NCHUNK_AGG = EPT // K
SB = 32
NSB_AGG = NCHUNK_AGG // SB
SB_DEG = 16
NCHUNK_DEG = (EP // (NS * NC)) // K
NSB_DEG = NCHUNK_DEG // SB_DEG
NPAD = 10240
ROWS_PT = NPAD // NS

_mesh = plsc.VectorSubcoreMesh(core_axis_name="c", subcore_axis_name="s")


def _deg_body(dst_hbm, out_hbm, dst_v, buf, acc, ssem):
    c = lax.axis_index("c")
    s = lax.axis_index("s")
    zeros16 = jnp.zeros((16,), jnp.float32)
    ones16 = jnp.ones((16,), jnp.float32)

    def fillrow(r, carry):
        for k in range(DH // 16):
            buf[0, r, pl.ds(k * 16, 16)] = zeros16
            buf[1, r, pl.ds(k * 16, 16)] = ones16
        return carry
    lax.fori_loop(0, K, fillrow, 0)

    row0 = s * ROWS_PT
    for q in range(ROWS_PT // K):
        pltpu.sync_copy(buf.at[0], acc.at[pl.ds(row0 + q * K, K)])
    plsc.subcore_barrier()

    wid = s * NC + c

    def sblock(sb, carry):
        pltpu.sync_copy(dst_hbm.at[wid, sb], dst_v)

        def step(j, carry2):
            pltpu.async_copy(buf.at[1], acc.at[dst_v.at[j]], ssem, add=True)
            pltpu.make_async_copy(buf.at[1], acc.at[dst_v.at[j]], ssem).wait()
            return carry2
        lax.fori_loop(0, SB_DEG, step, 0)
        return carry
    lax.fori_loop(0, NSB_DEG, sblock, 0)

    plsc.subcore_barrier()
    pltpu.sync_copy(acc.at[pl.ds(row0, ROWS_PT)],
                    out_hbm.at[pl.ds(c * NPAD + row0, ROWS_PT)])


_deg_call = functools.partial(
    pl.kernel,
    out_type=jax.ShapeDtypeStruct((NC * NPAD, DH), jnp.float32),
    mesh=_mesh,
    scratch_types=[
        pltpu.VMEM((SB_DEG, K), jnp.int32),
        pltpu.VMEM((2, K, DH), jnp.float32),
        pltpu.VMEM_SHARED((NPAD, DH), jnp.float32),
        pltpu.SemaphoreType.DMA,
    ],
)(_deg_body)


def _agg_body(src_hbm, dst_hbm, g_hbm, out_hbm, src_v, dst_v, buf, acc,
              gsem0, gsem1, ssem0, ssem1):
    c = lax.axis_index("c")
    s = lax.axis_index("s")
    gsems = (gsem0, gsem1)
    ssems = (ssem0, ssem1)
    zeros16 = jnp.zeros((16,), jnp.float32)

    def zrow(r, carry):
        for k in range(DH // 16):
            buf[0, r, pl.ds(k * 16, 16)] = zeros16
        return carry
    lax.fori_loop(0, K, zrow, 0)

    row0 = s * ROWS_PT
    for q in range(ROWS_PT // K):
        pltpu.sync_copy(buf.at[0], acc.at[pl.ds(row0 + q * K, K)])

    off = c * NPAD
    plsc.subcore_barrier()

    def gather(j, b, sem):
        pltpu.async_copy(g_hbm.at[src_v.at[j]], buf.at[b], sem)

    def gwait(j, b, sem):
        pltpu.make_async_copy(g_hbm.at[src_v.at[j]], buf.at[b], sem).wait()

    def sissue(j, b, sem):
        pltpu.async_copy(buf.at[b], acc.at[dst_v.at[j]], sem, add=True)

    def swait(j, b, sem):
        pltpu.make_async_copy(buf.at[b], acc.at[dst_v.at[j]], sem).wait()

    def sblock(sb, carry):
        pltpu.sync_copy(src_hbm.at[s, sb], src_v)
        pltpu.sync_copy(dst_hbm.at[s, sb], dst_v)

        def addoff(r, carry2):
            for k in range(K // 16):
                sl = pl.ds(k * 16, 16)
                src_v[r, sl] = src_v[r, sl] + off
            return carry2
        lax.fori_loop(0, SB, addoff, 0)

        gather(0, 0, gsems[0])
        gather(1, 1, gsems[1])

        def pair(i, carry2):
            j0 = i * 2
            gwait(j0, 0, gsems[0])
            sissue(j0, 0, ssems[0])
            gwait(j0 + 1, 1, gsems[1])
            sissue(j0 + 1, 1, ssems[1])
            swait(j0, 0, ssems[0])
            gather(jnp.minimum(j0 + 2, SB - 1), 0, gsems[0])
            swait(j0 + 1, 1, ssems[1])
            gather(jnp.minimum(j0 + 3, SB - 1), 1, gsems[1])
            return carry2
        lax.fori_loop(0, SB // 2, pair, 0)

        gwait(SB - 1, 0, gsems[0])
        gwait(SB - 1, 1, gsems[1])
        return carry
    lax.fori_loop(0, NSB_AGG, sblock, 0)

    plsc.subcore_barrier()
    pltpu.sync_copy(acc.at[pl.ds(row0, ROWS_PT)],
                    out_hbm.at[pl.ds(c * NPAD + row0, ROWS_PT)])


_agg_call = functools.partial(
    pl.kernel,
    out_type=jax.ShapeDtypeStruct((NC * NPAD, DH), jnp.float32),
    mesh=_mesh,
    scratch_types=[
        pltpu.VMEM((SB, K), jnp.int32),
        pltpu.VMEM((SB, K), jnp.int32),
        pltpu.VMEM((2, K, DH), jnp.float32),
        pltpu.VMEM_SHARED((NPAD, DH), jnp.float32),
        pltpu.SemaphoreType.DMA,
        pltpu.SemaphoreType.DMA,
        pltpu.SemaphoreType.DMA,
        pltpu.SemaphoreType.DMA,
    ],
)(_agg_body)


R = 400
NB = N // R


def _dinv_block(deg_ref):
    d = deg_ref[0, :, 0:1] + deg_ref[1, :, 0:1] + 1.0
    return jnp.where(d > 0, lax.rsqrt(jnp.maximum(d, 1e-12)), 0.0)


def _mm1_body(deg_ref, x_ref, w_ref, o_ref):
    dinv = _dinv_block(deg_ref)
    h = jnp.dot(x_ref[...], w_ref[...], preferred_element_type=jnp.float32)
    g = h * dinv
    o_ref[0] = g[:, :DH]
    o_ref[1] = g[:, DH:]


def _layer_body(deg_ref, agg_ref, g_ref, b_ref, w_ref, o_ref):
    dinv = _dinv_block(deg_ref)
    x0 = jnp.maximum((agg_ref[0] + g_ref[0]) * dinv + b_ref[0, :DH], 0.0)
    x1 = jnp.maximum((agg_ref[1] + g_ref[1]) * dinv + b_ref[0, DH:], 0.0)
    x = jnp.concatenate([x0, x1], axis=1)
    h = jnp.dot(x, w_ref[...], preferred_element_type=jnp.float32)
    g = h * dinv
    o_ref[0] = g[:, :DH]
    o_ref[1] = g[:, DH:]


def _final_body(deg_ref, agg_ref, g_ref, b_ref, o_ref):
    dinv = _dinv_block(deg_ref)
    x0 = (agg_ref[0] + g_ref[0]) * dinv + b_ref[0, :DH]
    x1 = (agg_ref[1] + g_ref[1]) * dinv + b_ref[0, DH:]
    o_ref[...] = jnp.concatenate([x0, x1], axis=1)


_deg_spec = pl.BlockSpec((2, R, DH), lambda j: (0, j, 0))
_half_spec = pl.BlockSpec((2, R, DH), lambda j: (0, j, 0))
_b_spec = pl.BlockSpec((1, 2 * DH), lambda j: (0, 0))


def _mm1(deg2, x, W1):
    return pl.pallas_call(
        _mm1_body,
        grid=(NB,),
        in_specs=[
            _deg_spec,
            pl.BlockSpec((R, 128), lambda j: (j, 0)),
            pl.BlockSpec((128, 2 * DH), lambda j: (0, 0)),
        ],
        out_specs=_half_spec,
        out_shape=jax.ShapeDtypeStruct((2, NPAD, DH), jnp.float32),
    )(deg2, x, W1)


def _layer(deg2, agg, g, b, W):
    return pl.pallas_call(
        _layer_body,
        grid=(NB,),
        in_specs=[
            _deg_spec,
            _half_spec,
            _half_spec,
            _b_spec,
            pl.BlockSpec((2 * DH, 2 * DH), lambda j: (0, 0)),
        ],
        out_specs=_half_spec,
        out_shape=jax.ShapeDtypeStruct((2, NPAD, DH), jnp.float32),
    )(deg2, agg, g, b, W)


def _final(deg2, agg, g, b):
    return pl.pallas_call(
        _final_body,
        grid=(NB,),
        in_specs=[_deg_spec, _half_spec, _half_spec, _b_spec],
        out_specs=pl.BlockSpec((R, 2 * DH), lambda j: (j, 0)),
        out_shape=jax.ShapeDtypeStruct((N, 2 * DH), jnp.float32),
    )(deg2, agg, g, b)


def kernel(x, edge_index, W1, b1, W2, b2, W3, b3):
    npad_e = EP - E
    pad_src = jnp.arange(npad_e, dtype=jnp.int32) % N
    pad_dst = N + jnp.arange(npad_e, dtype=jnp.int32) % (NPAD - N)
    src_p = jnp.concatenate([edge_index[0], pad_src])
    dst_p = jnp.concatenate([edge_index[1], pad_dst])
    src_r = src_p.reshape(NS, NSB_AGG, SB, K)
    dst_r = dst_p.reshape(NS, NSB_AGG, SB, K)
    b1r = b1.reshape(1, 2 * DH)
    b2r = b2.reshape(1, 2 * DH)
    b3r = b3.reshape(1, 2 * DH)

    dst_d = dst_p.reshape(NS * NC, NSB_DEG, SB_DEG, K)
    deg2 = _deg_call(dst_d).reshape(2, NPAD, DH)

    g1 = _mm1(deg2, x, W1)
    agg1 = _agg_call(src_r, dst_r, g1.reshape(NC * NPAD, DH)).reshape(2, NPAD, DH)
    g2 = _layer(deg2, agg1, g1, b1r, W2)
    agg2 = _agg_call(src_r, dst_r, g2.reshape(NC * NPAD, DH)).reshape(2, NPAD, DH)
    g3 = _layer(deg2, agg2, g2, b2r, W3)
    agg3 = _agg_call(src_r, dst_r, g3.reshape(NC * NPAD, DH)).reshape(2, NPAD, DH)
    return _final(deg2, agg3, g3, b3r)
